# baseline (speedup 1.0000x reference)
import functools

import jax
import jax.numpy as jnp
import numpy as np
from jax import lax
from jax.experimental import pallas as pl
from jax.experimental.pallas import tpu as pltpu
from jax.experimental.pallas import tpu_sc as plsc

N_IMG = 256
NC = 8
NK = 262144
CHUNK = 16384
UNROLL = 8


def _dft_mats():
    k = np.arange(N_IMG) - N_IMG // 2
    m = np.outer(k, k).astype(np.float64)
    f = np.exp(-2j * np.pi * m / N_IMG) / np.sqrt(N_IMG)
    return (jnp.asarray(f.real, dtype=jnp.float32),
            jnp.asarray(f.imag, dtype=jnp.float32))


def _fft_body(fr_ref, fi_ref, x_ref, out_ref):
    x = x_ref[0]
    fr = fr_ref[...]
    fi = fi_ref[...]
    ar = jnp.dot(fr, x, preferred_element_type=jnp.float32)
    ai = jnp.dot(fi, x, preferred_element_type=jnp.float32)
    out_ref[0, 0, 0] = (jnp.dot(ar, fr, preferred_element_type=jnp.float32)
                        - jnp.dot(ai, fi, preferred_element_type=jnp.float32))
    out_ref[0, 1, 0] = (jnp.dot(ar, fi, preferred_element_type=jnp.float32)
                        + jnp.dot(ai, fr, preferred_element_type=jnp.float32))


def _centered_fft2(img_flat, nb):
    fr, fi = _dft_mats()
    nimg = img_flat.shape[0]
    return pl.pallas_call(
        _fft_body,
        grid=(nimg,),
        in_specs=[
            pl.BlockSpec((N_IMG, N_IMG), lambda b: (0, 0)),
            pl.BlockSpec((N_IMG, N_IMG), lambda b: (0, 0)),
            pl.BlockSpec((1, N_IMG, N_IMG), lambda b: (b, 0, 0)),
        ],
        out_specs=pl.BlockSpec((1, 2, 1, N_IMG, N_IMG),
                               lambda b: (b // NC, 0, b % NC, 0, 0)),
        out_shape=jax.ShapeDtypeStruct((nb, 2, NC, N_IMG, N_IMG), jnp.float32),
    )(fr, fi, img_flat)


def _make_sc_gather(n):
    mesh = plsc.VectorSubcoreMesh(core_axis_name="c", subcore_axis_name="s")
    khalf = NK // 2

    @functools.partial(
        pl.kernel,
        mesh=mesh,
        out_type=(jax.ShapeDtypeStruct((NC, NK), jnp.float32),
                  jax.ShapeDtypeStruct((NC, NK), jnp.float32)),
        scratch_types=[
            pltpu.VMEM((N_IMG * N_IMG,), jnp.float32),
            pltpu.VMEM((CHUNK,), jnp.int32),
            pltpu.VMEM((CHUNK,), jnp.float32),
        ],
        compiler_params=pltpu.CompilerParams(needs_layout_passes=False),
    )
    def gather_kernel(grids_hbm, idx_hbm, re_hbm, im_hbm, grid_v, idx_v, out_v):
        wid = lax.axis_index("s") * 2 + lax.axis_index("c")
        p = wid // 2
        h = wid % 2
        isim = p // NC
        c = p % NC
        pltpu.sync_copy(grids_hbm.at[n, isim, c], grid_v)

        def chunk_body(kc, _):
            base = h * khalf + kc * CHUNK
            pltpu.sync_copy(idx_hbm.at[n, pl.ds(base, CHUNK)], idx_v)

            def g_body(j, _):
                b0 = j * (16 * UNROLL)
                for u in range(UNROLL):
                    off = b0 + u * 16
                    iv = idx_v[pl.ds(off, 16)]
                    out_v[pl.ds(off, 16)] = plsc.load_gather(grid_v, [iv])
                return 0

            lax.fori_loop(0, CHUNK // (16 * UNROLL), g_body, 0)

            @pl.when(isim == 0)
            def _():
                pltpu.sync_copy(out_v, re_hbm.at[c, pl.ds(base, CHUNK)])

            @pl.when(isim == 1)
            def _():
                pltpu.sync_copy(out_v, im_hbm.at[c, pl.ds(base, CHUNK)])

            return 0

        lax.fori_loop(0, khalf // CHUNK, chunk_body, 0)

    return gather_kernel


def kernel(img, trj):
    nb, nc = img.shape[0], img.shape[1]
    img_flat = img.reshape(nb * nc, N_IMG, N_IMG)
    grids = _centered_fft2(img_flat, nb)
    grids = grids.reshape(nb, 2, nc, N_IMG * N_IMG)
    idx = trj[..., 0] * N_IMG + trj[..., 1]
    outs = []
    for n in range(nb):
        re_n, im_n = _make_sc_gather(n)(grids, idx)
        outs.append(lax.optimization_barrier(lax.complex(re_n, im_n)))
    return jnp.stack(outs, axis=0)


# speedup vs baseline: 1.1172x; 1.1172x over previous
import functools

import jax
import jax.numpy as jnp
import numpy as np
from jax import lax
from jax.experimental import pallas as pl
from jax.experimental.pallas import tpu as pltpu
from jax.experimental.pallas import tpu_sc as plsc

N_IMG = 256
NC = 8
NB = 4
NK = 262144
CHUNK = 16384
UNROLL = 8


def _dft_mats():
    k = np.arange(N_IMG) - N_IMG // 2
    m = np.outer(k, k).astype(np.float64)
    f = np.exp(-2j * np.pi * m / N_IMG) / np.sqrt(N_IMG)
    return (jnp.asarray(f.real, dtype=jnp.float32),
            jnp.asarray(f.imag, dtype=jnp.float32))


def _fft_body(fr_ref, fi_ref, x_ref, out_ref):
    x = x_ref[0]
    fr = fr_ref[...]
    fi = fi_ref[...]
    ar = jnp.dot(fr, x, preferred_element_type=jnp.float32)
    ai = jnp.dot(fi, x, preferred_element_type=jnp.float32)
    out_ref[0, 0] = (jnp.dot(ar, fr, preferred_element_type=jnp.float32)
                     - jnp.dot(ai, fi, preferred_element_type=jnp.float32))
    out_ref[1, 0] = (jnp.dot(ar, fi, preferred_element_type=jnp.float32)
                     + jnp.dot(ai, fr, preferred_element_type=jnp.float32))


def _centered_fft2(img_flat):
    fr, fi = _dft_mats()
    nimg = img_flat.shape[0]
    return pl.pallas_call(
        _fft_body,
        grid=(nimg,),
        in_specs=[
            pl.BlockSpec((N_IMG, N_IMG), lambda b: (0, 0)),
            pl.BlockSpec((N_IMG, N_IMG), lambda b: (0, 0)),
            pl.BlockSpec((1, N_IMG, N_IMG), lambda b: (b, 0, 0)),
        ],
        out_specs=pl.BlockSpec((2, 1, N_IMG, N_IMG), lambda b: (0, b, 0, 0)),
        out_shape=jax.ShapeDtypeStruct((2, nimg, N_IMG, N_IMG), jnp.float32),
    )(fr, fi, img_flat)


def _sc_gather(grids, idx):
    mesh = plsc.VectorSubcoreMesh(core_axis_name="c", subcore_axis_name="s")

    @functools.partial(
        pl.kernel,
        mesh=mesh,
        out_type=(jax.ShapeDtypeStruct((NB * NC, NK), jnp.float32),
                  jax.ShapeDtypeStruct((NB * NC, NK), jnp.float32)),
        scratch_types=[
            pltpu.VMEM((N_IMG * N_IMG,), jnp.float32),
            pltpu.VMEM((CHUNK,), jnp.int32),
            pltpu.VMEM((CHUNK,), jnp.float32),
        ],
        compiler_params=pltpu.CompilerParams(needs_layout_passes=False),
    )
    def gather_kernel(grids_hbm, idx_hbm, re_hbm, im_hbm, grid_v, idx_v, out_v):
        wid = lax.axis_index("s") * 2 + lax.axis_index("c")
        for isim in range(2):
            q = wid
            n = q // NC
            pltpu.sync_copy(grids_hbm.at[isim * 32 + q], grid_v)

            def chunk_body(kc, _, n=n, q=q, isim=isim):
                base = kc * CHUNK
                pltpu.sync_copy(idx_hbm.at[n, pl.ds(base, CHUNK)], idx_v)

                def g_body(j, _):
                    b0 = j * (16 * UNROLL)
                    for u in range(UNROLL):
                        off = b0 + u * 16
                        iv = idx_v[pl.ds(off, 16)]
                        out_v[pl.ds(off, 16)] = plsc.load_gather(grid_v, [iv])
                    return 0

                lax.fori_loop(0, CHUNK // (16 * UNROLL), g_body, 0)
                dst = re_hbm if isim == 0 else im_hbm
                pltpu.sync_copy(out_v, dst.at[q, pl.ds(base, CHUNK)])
                return 0

            lax.fori_loop(0, NK // CHUNK, chunk_body, 0)

    return gather_kernel(grids, idx)


def kernel(img, trj):
    nb, nc = img.shape[0], img.shape[1]
    img_flat = img.reshape(nb * nc, N_IMG, N_IMG)
    grids = _centered_fft2(img_flat)
    grids = grids.reshape(2 * nb * nc, N_IMG * N_IMG)
    idx = trj[..., 0] * N_IMG + trj[..., 1]
    re, im = _sc_gather(grids, idx)
    return lax.complex(re, im).reshape(nb, nc, NK)


# speedup vs baseline: 1.1536x; 1.0326x over previous
import functools

import jax
import jax.numpy as jnp
import numpy as np
from jax import lax
from jax.experimental import pallas as pl
from jax.experimental.pallas import tpu as pltpu
from jax.experimental.pallas import tpu_sc as plsc

N_IMG = 256
NC = 8
NB = 4
NK = 262144
CHUNK = 8192
UNROLL = 8


def _dft_mats():
    k = np.arange(N_IMG) - N_IMG // 2
    m = np.outer(k, k).astype(np.float64)
    f = np.exp(-2j * np.pi * m / N_IMG) / np.sqrt(N_IMG)
    return (jnp.asarray(f.real, dtype=jnp.float32),
            jnp.asarray(f.imag, dtype=jnp.float32))


def _fft_body(fr_ref, fi_ref, x_ref, out_ref):
    x = x_ref[0]
    fr = fr_ref[...]
    fi = fi_ref[...]
    ar = jnp.dot(fr, x, preferred_element_type=jnp.float32)
    ai = jnp.dot(fi, x, preferred_element_type=jnp.float32)
    out_ref[0, 0] = (jnp.dot(ar, fr, preferred_element_type=jnp.float32)
                     - jnp.dot(ai, fi, preferred_element_type=jnp.float32))
    out_ref[1, 0] = (jnp.dot(ar, fi, preferred_element_type=jnp.float32)
                     + jnp.dot(ai, fr, preferred_element_type=jnp.float32))


def _centered_fft2(img_flat):
    fr, fi = _dft_mats()
    nimg = img_flat.shape[0]
    return pl.pallas_call(
        _fft_body,
        grid=(nimg,),
        in_specs=[
            pl.BlockSpec((N_IMG, N_IMG), lambda b: (0, 0)),
            pl.BlockSpec((N_IMG, N_IMG), lambda b: (0, 0)),
            pl.BlockSpec((1, N_IMG, N_IMG), lambda b: (b, 0, 0)),
        ],
        out_specs=pl.BlockSpec((2, 1, N_IMG, N_IMG), lambda b: (0, b, 0, 0)),
        out_shape=jax.ShapeDtypeStruct((2, nimg, N_IMG, N_IMG), jnp.float32),
    )(fr, fi, img_flat)


def _sc_gather(grids, idx):
    mesh = plsc.VectorSubcoreMesh(core_axis_name="c", subcore_axis_name="s")

    nchunks = NK // CHUNK

    @functools.partial(
        pl.kernel,
        mesh=mesh,
        out_type=(jax.ShapeDtypeStruct((NB * NC, NK), jnp.float32),
                  jax.ShapeDtypeStruct((NB * NC, NK), jnp.float32)),
        scratch_types=[
            pltpu.VMEM((N_IMG * N_IMG,), jnp.float32),
            pltpu.VMEM((2, CHUNK), jnp.int32),
            pltpu.VMEM((2, CHUNK), jnp.float32),
            pltpu.SemaphoreType.DMA,
            pltpu.SemaphoreType.DMA,
            pltpu.SemaphoreType.DMA,
            pltpu.SemaphoreType.DMA,
        ],
        compiler_params=pltpu.CompilerParams(needs_layout_passes=False),
    )
    def gather_kernel(grids_hbm, idx_hbm, re_hbm, im_hbm,
                      grid_v, idx_v, out_v, isem0, isem1, osem0, osem1):
        wid = lax.axis_index("s") * 2 + lax.axis_index("c")
        isems = (isem0, isem1)
        osems = (osem0, osem1)
        q = wid
        n = q // NC
        for isim in range(2):
            pltpu.sync_copy(grids_hbm.at[isim * 32 + q], grid_v)
            dst = re_hbm if isim == 0 else im_hbm

            idx_h = pltpu.async_copy(
                idx_hbm.at[n, pl.ds(0, CHUNK)], idx_v.at[0], isems[0])
            out_hs = [None, None]
            for kc in range(nchunks):
                b = kc % 2
                nxt = None
                if kc + 1 < nchunks:
                    nxt = pltpu.async_copy(
                        idx_hbm.at[n, pl.ds((kc + 1) * CHUNK, CHUNK)],
                        idx_v.at[1 - b], isems[1 - b])
                idx_h.wait()
                if out_hs[b] is not None:
                    out_hs[b].wait()

                def g_body(j, _, b=b):
                    b0 = j * (16 * UNROLL)
                    for u in range(UNROLL):
                        off = b0 + u * 16
                        iv = idx_v[b, pl.ds(off, 16)]
                        out_v[b, pl.ds(off, 16)] = plsc.load_gather(
                            grid_v, [iv])
                    return 0

                lax.fori_loop(0, CHUNK // (16 * UNROLL), g_body, 0)
                out_hs[b] = pltpu.async_copy(
                    out_v.at[b], dst.at[q, pl.ds(kc * CHUNK, CHUNK)], osems[b])
                idx_h = nxt
            for h in out_hs:
                if h is not None:
                    h.wait()

    return gather_kernel(grids, idx)


def kernel(img, trj):
    nb, nc = img.shape[0], img.shape[1]
    img_flat = img.reshape(nb * nc, N_IMG, N_IMG)
    grids = _centered_fft2(img_flat)
    grids = grids.reshape(2 * nb * nc, N_IMG * N_IMG)
    idx = trj[..., 0] * N_IMG + trj[..., 1]
    re, im = _sc_gather(grids, idx)
    return lax.complex(re, im).reshape(nb, nc, NK)


# speedup vs baseline: 1.3072x; 1.1331x over previous
import functools

import jax
import jax.numpy as jnp
import numpy as np
from jax import lax
from jax.experimental import pallas as pl
from jax.experimental.pallas import tpu as pltpu
from jax.experimental.pallas import tpu_sc as plsc

N_IMG = 256
NC = 8
NB = 4
NK = 262144
CHUNK = 8192
UNROLL = 8


def _dft_mats():
    k = np.arange(N_IMG) - N_IMG // 2
    m = np.outer(k, k).astype(np.float64)
    f = np.exp(-2j * np.pi * m / N_IMG) / np.sqrt(N_IMG)
    return (jnp.asarray(f.real, dtype=jnp.float32),
            jnp.asarray(f.imag, dtype=jnp.float32))


def _fft_body(fr_ref, fi_ref, x_ref, out_ref):
    x = x_ref[0]
    fr = fr_ref[...]
    fi = fi_ref[...]
    ar = jnp.dot(fr, x, preferred_element_type=jnp.float32)
    ai = jnp.dot(fi, x, preferred_element_type=jnp.float32)
    out_ref[0, 0] = (jnp.dot(ar, fr, preferred_element_type=jnp.float32)
                     - jnp.dot(ai, fi, preferred_element_type=jnp.float32))
    out_ref[1, 0] = (jnp.dot(ar, fi, preferred_element_type=jnp.float32)
                     + jnp.dot(ai, fr, preferred_element_type=jnp.float32))


def _centered_fft2(img_flat):
    fr, fi = _dft_mats()
    nimg = img_flat.shape[0]
    return pl.pallas_call(
        _fft_body,
        grid=(nimg,),
        in_specs=[
            pl.BlockSpec((N_IMG, N_IMG), lambda b: (0, 0)),
            pl.BlockSpec((N_IMG, N_IMG), lambda b: (0, 0)),
            pl.BlockSpec((1, N_IMG, N_IMG), lambda b: (b, 0, 0)),
        ],
        out_specs=pl.BlockSpec((2, 1, N_IMG, N_IMG), lambda b: (0, b, 0, 0)),
        out_shape=jax.ShapeDtypeStruct((2, nimg, N_IMG, N_IMG), jnp.float32),
    )(fr, fi, img_flat)


def _sc_gather(grids, idx):
    mesh = plsc.VectorSubcoreMesh(core_axis_name="c", subcore_axis_name="s")

    nchunks = NK // CHUNK

    @functools.partial(
        pl.kernel,
        mesh=mesh,
        out_type=(jax.ShapeDtypeStruct((NB * NC, NK), jnp.float32),
                  jax.ShapeDtypeStruct((NB * NC, NK), jnp.float32)),
        scratch_types=[
            pltpu.VMEM((N_IMG * N_IMG,), jnp.float32),
            pltpu.VMEM((2, CHUNK), jnp.int32),
            pltpu.VMEM((2, CHUNK), jnp.float32),
            pltpu.SemaphoreType.DMA,
            pltpu.SemaphoreType.DMA,
            pltpu.SemaphoreType.DMA,
            pltpu.SemaphoreType.DMA,
        ],
        compiler_params=pltpu.CompilerParams(needs_layout_passes=False),
    )
    def gather_kernel(grids_hbm, idx_hbm, re_hbm, im_hbm,
                      grid_v, idx_v, out_v, isem0, isem1, osem0, osem1):
        wid = lax.axis_index("s") * 2 + lax.axis_index("c")
        isems = (isem0, isem1)
        osems = (osem0, osem1)
        q = wid
        n = q // NC
        for isim in range(2):
            pltpu.sync_copy(grids_hbm.at[isim * 32 + q], grid_v)
            dst = re_hbm if isim == 0 else im_hbm

            idx_h = pltpu.async_copy(
                idx_hbm.at[n, pl.ds(0, CHUNK)], idx_v.at[0], isems[0])
            out_hs = [None, None]
            for kc in range(nchunks):
                b = kc % 2
                nxt = None
                if kc + 1 < nchunks:
                    nxt = pltpu.async_copy(
                        idx_hbm.at[n, pl.ds((kc + 1) * CHUNK, CHUNK)],
                        idx_v.at[1 - b], isems[1 - b])
                idx_h.wait()
                if out_hs[b] is not None:
                    out_hs[b].wait()

                @plsc.parallel_loop(0, CHUNK, 16, unroll=UNROLL)
                def g_body(off, b=b):
                    iv = idx_v[b, pl.ds(off, 16)]
                    out_v[b, pl.ds(off, 16)] = plsc.load_gather(grid_v, [iv])
                out_hs[b] = pltpu.async_copy(
                    out_v.at[b], dst.at[q, pl.ds(kc * CHUNK, CHUNK)], osems[b])
                idx_h = nxt
            for h in out_hs:
                if h is not None:
                    h.wait()

    return gather_kernel(grids, idx)


def kernel(img, trj):
    nb, nc = img.shape[0], img.shape[1]
    img_flat = img.reshape(nb * nc, N_IMG, N_IMG)
    grids = _centered_fft2(img_flat)
    grids = grids.reshape(2 * nb * nc, N_IMG * N_IMG)
    idx = trj[..., 0] * N_IMG + trj[..., 1]
    re, im = _sc_gather(grids, idx)
    return lax.complex(re, im).reshape(nb, nc, NK)
